# baseline (speedup 1.0000x reference)
import functools

import jax
import jax.numpy as jnp
from jax import lax
from jax.experimental import pallas as pl
from jax.experimental.pallas import tpu as pltpu
from jax.experimental.pallas import tpu_sc as plsc

_NC = 2
_NS = 16
_NW = _NC * _NS


def _gather_rows(table, idx):
    V, D = table.shape
    B = idx.shape[0]
    b_per_w = B // _NW
    mesh = plsc.VectorSubcoreMesh(core_axis_name="c", subcore_axis_name="s")

    @functools.partial(
        pl.kernel,
        mesh=mesh,
        out_type=jax.ShapeDtypeStruct((B, D), jnp.float32),
        scratch_types=[
            pltpu.VMEM((b_per_w,), jnp.int32),
            pltpu.VMEM((b_per_w, D), jnp.float32),
            pltpu.SemaphoreType.DMA,
        ],
    )
    def k(table_hbm, idx_hbm, out_hbm, idx_v, rows_v, sem):
        wid = lax.axis_index("s") * _NC + lax.axis_index("c")
        base = wid * b_per_w
        pltpu.sync_copy(idx_hbm.at[pl.ds(base, b_per_w)], idx_v)
        pltpu.async_copy(table_hbm.at[idx_v], rows_v, sem).wait()
        pltpu.sync_copy(rows_v, out_hbm.at[pl.ds(base, b_per_w)])

    return k(table, idx)


def _copy_table(memory):
    M, H = memory.shape
    MBLK = 2000

    def body(mem_ref, out_ref):
        out_ref[...] = mem_ref[...]

    return pl.pallas_call(
        body,
        grid=(M // MBLK,),
        in_specs=[pl.BlockSpec((MBLK, H), lambda i: (i, 0))],
        out_specs=pl.BlockSpec((MBLK, H), lambda i: (i, 0)),
        out_shape=jax.ShapeDtypeStruct((M, H), jnp.float32),
    )(memory)


def _dense_update(nf, nm, ef, W1, b1, W2, b2, W_ih, W_hh, b_ih, b_hh):
    B, H = nm.shape
    G = 32
    BLK = B // G

    b1r = b1.reshape(1, -1)
    b2r = b2.reshape(1, -1)
    b_ihr = b_ih.reshape(1, -1)
    b_hhr = b_hh.reshape(1, -1)

    def body(nf_ref, nm_ref, ef_ref, W1_ref, b1_ref, W2_ref, b2_ref,
             W_ih_ref, W_hh_ref, b_ih_ref, b_hh_ref, out_ref):
        nf_b = nf_ref[...]
        nm_b = nm_ref[...]
        ef_b = ef_ref[...]
        x = jnp.concatenate([nf_b, nm_b, ef_b], axis=1)
        h1 = jnp.maximum(
            jnp.dot(x, W1_ref[...], preferred_element_type=jnp.float32)
            + b1_ref[...], 0.0)
        msgs = jnp.dot(h1, W2_ref[...], preferred_element_type=jnp.float32) + b2_ref[...]
        gx = lax.dot_general(msgs, W_ih_ref[...], (((1,), (1,)), ((), ())),
                             preferred_element_type=jnp.float32) + b_ih_ref[...]
        gh = lax.dot_general(nm_b, W_hh_ref[...], (((1,), (1,)), ((), ())),
                             preferred_element_type=jnp.float32) + b_hh_ref[...]
        r = jax.nn.sigmoid(gx[:, :H] + gh[:, :H])
        z = jax.nn.sigmoid(gx[:, H:2 * H] + gh[:, H:2 * H])
        n = jnp.tanh(gx[:, 2 * H:] + r * gh[:, 2 * H:])
        out_ref[...] = (1.0 - z) * n + z * nm_b

    full = lambda shape: pl.BlockSpec(shape, lambda i: (0,) * len(shape))
    return pl.pallas_call(
        body,
        grid=(G,),
        in_specs=[
            pl.BlockSpec((BLK, H), lambda i: (i, 0)),
            pl.BlockSpec((BLK, H), lambda i: (i, 0)),
            pl.BlockSpec((BLK, H), lambda i: (i, 0)),
            full(W1.shape), full(b1r.shape), full(W2.shape), full(b2r.shape),
            full(W_ih.shape), full(W_hh.shape), full(b_ihr.shape), full(b_hhr.shape),
        ],
        out_specs=pl.BlockSpec((BLK, H), lambda i: (i, 0)),
        out_shape=jax.ShapeDtypeStruct((B, H), jnp.float32),
    )(nf, nm, ef, W1, b1r, W2, b2r, W_ih, W_hh, b_ihr, b_hhr)


_WIN = 3328
_SENT = 3200
_GRP = 128
_LCAP = 3456


def _index_winners(idx, M):
    B = idx.shape[0]
    SLAB_LO = (M // _NW) & ~7
    EXTRA = (M - SLAB_LO * _NW) // 8
    SLAB_HI = SLAB_LO + 8
    NIV = B // 16
    mesh = plsc.VectorSubcoreMesh(core_axis_name="c", subcore_axis_name="s")

    @functools.partial(
        pl.kernel,
        mesh=mesh,
        out_type=(
            jax.ShapeDtypeStruct((_NW, _LCAP), jnp.int32),
            jax.ShapeDtypeStruct((_NW, _LCAP), jnp.int32),
            jax.ShapeDtypeStruct((_NW, 16), jnp.int32),
        ),
        scratch_types=[
            pltpu.VMEM((B + 16,), jnp.int32),
            pltpu.VMEM((B + 48,), jnp.int32),
            pltpu.VMEM((_WIN,), jnp.int32),
            pltpu.VMEM((_LCAP,), jnp.int32),
            pltpu.VMEM((_LCAP,), jnp.int32),
            pltpu.VMEM((16,), jnp.int32),
        ],
        compiler_params=pltpu.CompilerParams(
            needs_layout_passes=False, use_tc_tiling_on_sc=True),
    )
    def k(idx_hbm, pf_hbm, gf_hbm, cnt_hbm,
          idx_v, comp_v, win_v, pfin_v, gfin1_v, cnt_v):
        iota = lax.iota(jnp.int32, 16)
        wid = lax.axis_index("s") * _NC + lax.axis_index("c")
        lo = wid * SLAB_LO + 8 * jnp.minimum(wid, EXTRA)
        has_extra = wid < EXTRA
        nrows = jnp.where(has_extra, SLAB_HI, SLAB_LO)
        WIN = _WIN
        SENT = _SENT
        LIST = _LCAP - _GRP
        pltpu.sync_copy(idx_hbm, idx_v.at[pl.ds(0, B)])
        neg1 = jnp.full((16,), -1, jnp.int32)

        def init_body(j, _):
            win_v[pl.ds(j * 16, 16)] = neg1
            return 0
        lax.fori_loop(0, WIN // 16, init_body, 0)

        hi = lo + nrows

        one = jnp.full((16,), 1, jnp.int32)
        zero = jnp.full((16,), 0, jnp.int32)

        def filt_body(i, off):
            v = idx_v[pl.ds(i * 16, 16)]
            inm = (v >= lo) & (v < hi)
            inm_i = jnp.where(inm, one, zero)
            comp = (v - lo) * 16384 + (i * 16 + iota)
            cs = plsc.cumsum(inm_i)
            pos = jnp.where(inm, off + cs - 1, B + 32 + iota)
            plsc.store_scatter(comp_v, [pos], comp)
            return off + jnp.sum(inm_i)
        off = lax.fori_loop(0, NIV, filt_body, jnp.int32(0))
        comp_v[pl.ds(off, 16)] = jnp.full((16,), SENT * 16384, jnp.int32)

        lane_masks = [iota == l for l in range(16)]

        def dedup_body(j, _):
            cvec = comp_v[pl.ds(j * 16, 16)]
            loc = lax.shift_right_logical(cvec, 14)
            p = cvec & 16383
            for lm in lane_masks:
                loc_safe = jnp.where(lm, loc, SENT + 16 + iota)
                plsc.store_scatter(win_v, [loc_safe], p)
            return 0
        lax.fori_loop(0, (off + 15) // 16, dedup_body, 0)

        def scan_body(j, cnt):
            w = win_v[pl.ds(j * 16, 16)]
            mk = w >= 0
            mk_i = jnp.where(mk, one, zero)
            g = (lo + j * 16) + iota
            cs = plsc.cumsum(mk_i)
            pos = jnp.where(mk, cnt + cs - 1, LIST + iota)
            plsc.store_scatter(pfin_v, [pos], w)
            plsc.store_scatter(gfin1_v, [pos], g)
            return cnt + jnp.sum(mk_i)
        cnt = lax.fori_loop(0, (SLAB_HI + 15) // 16, scan_body, jnp.int32(0))

        @pl.when(cnt > 0)
        def _():
            zeros = jnp.zeros((16,), jnp.int32)
            p0 = pfin_v[pl.ds(0, 16)].at[zeros].get(mode="promise_in_bounds")
            g0 = gfin1_v[pl.ds(0, 16)].at[zeros].get(mode="promise_in_bounds")
            for t in range(_GRP // 16):
                pfin_v[pl.ds(cnt + t * 16, 16)] = p0
                gfin1_v[pl.ds(cnt + t * 16, 16)] = g0

        cnt_v[pl.ds(0, 16)] = jnp.where(iota == 0, cnt, 0)
        pltpu.sync_copy(pfin_v, pf_hbm.at[wid])
        pltpu.sync_copy(gfin1_v, gf_hbm.at[wid])
        pltpu.sync_copy(cnt_v, cnt_hbm.at[wid])

    return k(idx)


def _scatter_rows(pf, gf, cnts, upd, out_ref, D):
    mesh = plsc.VectorSubcoreMesh(core_axis_name="c", subcore_axis_name="s")

    @functools.partial(
        pl.kernel,
        mesh=mesh,
        out_type=(),
        scratch_types=[
            pltpu.VMEM((_LCAP,), jnp.int32),
            pltpu.VMEM((_LCAP,), jnp.int32),
            pltpu.VMEM((8, 16, D), jnp.float32),
            pltpu.VMEM((16,), jnp.int32),
            pltpu.SemaphoreType.DMA,
            pltpu.SemaphoreType.DMA,
        ],
        compiler_params=pltpu.CompilerParams(
            needs_layout_passes=False, use_tc_tiling_on_sc=True),
    )
    def k(pf_hbm, gf_hbm, cnt_hbm, upd_hbm, out_hbm,
          pfin_v, gfin_v, ring, cnt_v, sem_g, sem_s):
        wid = lax.axis_index("s") * _NC + lax.axis_index("c")
        pltpu.sync_copy(pf_hbm.at[wid], pfin_v)
        pltpu.sync_copy(gf_hbm.at[wid], gfin_v)
        pltpu.sync_copy(cnt_hbm.at[wid], cnt_v)
        cnt = jnp.sum(cnt_v[pl.ds(0, 16)])

        @pl.when(cnt > 0)
        def _():
            def grp_body(g, _):
                base = g * _GRP
                descs = []
                for j in range(8):
                    pv = pfin_v[pl.ds(base + j * 16, 16)]
                    descs.append(
                        pltpu.async_copy(upd_hbm.at[pv], ring.at[j], sem_g))
                sdescs = []
                for j in range(8):
                    descs[j].wait()
                    gv = gfin_v[pl.ds(base + j * 16, 16)]
                    sdescs.append(
                        pltpu.async_copy(ring.at[j], out_hbm.at[gv], sem_s))
                for j in range(8):
                    sdescs[j].wait()
                return 0
            lax.fori_loop(0, (cnt + _GRP - 1) // _GRP, grp_body, 0)

    k(pf, gf, cnts, upd, out_ref)


def kernel(node_idxs, node_features, edge_features, timestamps, memory, messages, last_update, W1, b1, W2, b2, W_ih, W_hh, b_ih, b_hh):
    M, H = memory.shape
    node_memory = _gather_rows(memory, node_idxs)
    pf, gf, cnts = _index_winners(node_idxs, M)
    mem_copy = _copy_table(memory)
    updated = _dense_update(node_features, node_memory, edge_features,
                            W1, b1, W2, b2, W_ih, W_hh, b_ih, b_hh)
    out_ref = jax.new_ref(mem_copy)
    _scatter_rows(pf, gf, cnts, updated, out_ref, H)
    return out_ref[...]


# speedup vs baseline: 1.0818x; 1.0818x over previous
import functools

import jax
import jax.numpy as jnp
from jax import lax
from jax.experimental import pallas as pl
from jax.experimental.pallas import tpu as pltpu
from jax.experimental.pallas import tpu_sc as plsc

_NC = 2
_NS = 16
_NW = _NC * _NS


def _gather_rows(table, idx):
    V, D = table.shape
    B = idx.shape[0]
    b_per_w = B // _NW
    mesh = plsc.VectorSubcoreMesh(core_axis_name="c", subcore_axis_name="s")

    @functools.partial(
        pl.kernel,
        mesh=mesh,
        out_type=jax.ShapeDtypeStruct((B, D), jnp.float32),
        scratch_types=[
            pltpu.VMEM((b_per_w,), jnp.int32),
            pltpu.VMEM((b_per_w, D), jnp.float32),
            pltpu.SemaphoreType.DMA,
        ],
    )
    def k(table_hbm, idx_hbm, out_hbm, idx_v, rows_v, sem):
        wid = lax.axis_index("s") * _NC + lax.axis_index("c")
        base = wid * b_per_w
        pltpu.sync_copy(idx_hbm.at[pl.ds(base, b_per_w)], idx_v)
        pltpu.async_copy(table_hbm.at[idx_v], rows_v, sem).wait()
        pltpu.sync_copy(rows_v, out_hbm.at[pl.ds(base, b_per_w)])

    return k(table, idx)


def _copy_table(memory):
    M, H = memory.shape
    MBLK = 2000

    def body(mem_ref, out_ref):
        out_ref[...] = mem_ref[...]

    return pl.pallas_call(
        body,
        grid=(M // MBLK,),
        in_specs=[pl.BlockSpec((MBLK, H), lambda i: (i, 0))],
        out_specs=pl.BlockSpec((MBLK, H), lambda i: (i, 0)),
        out_shape=jax.ShapeDtypeStruct((M, H), jnp.float32),
    )(memory)


def _dense_update(nf, nm, ef, W1, b1, W2, b2, W_ih, W_hh, b_ih, b_hh):
    B, H = nm.shape
    G = 32
    BLK = B // G

    b1r = b1.reshape(1, -1)
    b2r = b2.reshape(1, -1)
    b_ihr = b_ih.reshape(1, -1)
    b_hhr = b_hh.reshape(1, -1)

    def body(nf_ref, nm_ref, ef_ref, W1_ref, b1_ref, W2_ref, b2_ref,
             W_ih_ref, W_hh_ref, b_ih_ref, b_hh_ref, out_ref):
        nf_b = nf_ref[...]
        nm_b = nm_ref[...]
        ef_b = ef_ref[...]
        x = jnp.concatenate([nf_b, nm_b, ef_b], axis=1)
        h1 = jnp.maximum(
            jnp.dot(x, W1_ref[...], preferred_element_type=jnp.float32)
            + b1_ref[...], 0.0)
        msgs = jnp.dot(h1, W2_ref[...], preferred_element_type=jnp.float32) + b2_ref[...]
        gx = lax.dot_general(msgs, W_ih_ref[...], (((1,), (1,)), ((), ())),
                             preferred_element_type=jnp.float32) + b_ih_ref[...]
        gh = lax.dot_general(nm_b, W_hh_ref[...], (((1,), (1,)), ((), ())),
                             preferred_element_type=jnp.float32) + b_hh_ref[...]
        r = jax.nn.sigmoid(gx[:, :H] + gh[:, :H])
        z = jax.nn.sigmoid(gx[:, H:2 * H] + gh[:, H:2 * H])
        n = jnp.tanh(gx[:, 2 * H:] + r * gh[:, 2 * H:])
        out_ref[...] = (1.0 - z) * n + z * nm_b

    full = lambda shape: pl.BlockSpec(shape, lambda i: (0,) * len(shape))
    return pl.pallas_call(
        body,
        grid=(G,),
        in_specs=[
            pl.BlockSpec((BLK, H), lambda i: (i, 0)),
            pl.BlockSpec((BLK, H), lambda i: (i, 0)),
            pl.BlockSpec((BLK, H), lambda i: (i, 0)),
            full(W1.shape), full(b1r.shape), full(W2.shape), full(b2r.shape),
            full(W_ih.shape), full(W_hh.shape), full(b_ihr.shape), full(b_hhr.shape),
        ],
        out_specs=pl.BlockSpec((BLK, H), lambda i: (i, 0)),
        out_shape=jax.ShapeDtypeStruct((B, H), jnp.float32),
    )(nf, nm, ef, W1, b1r, W2, b2r, W_ih, W_hh, b_ihr, b_hhr)


_WIN = 3328
_SENT = 3200
_GRP = 768
_LCAP = 4096


def _index_winners(idx, M):
    B = idx.shape[0]
    SLAB_LO = (M // _NW) & ~7
    EXTRA = (M - SLAB_LO * _NW) // 8
    SLAB_HI = SLAB_LO + 8
    NIV = B // 16
    mesh = plsc.VectorSubcoreMesh(core_axis_name="c", subcore_axis_name="s")

    @functools.partial(
        pl.kernel,
        mesh=mesh,
        out_type=(
            jax.ShapeDtypeStruct((_NW, _LCAP), jnp.int32),
            jax.ShapeDtypeStruct((_NW, _LCAP), jnp.int32),
            jax.ShapeDtypeStruct((_NW, 16), jnp.int32),
        ),
        scratch_types=[
            pltpu.VMEM((B + 16,), jnp.int32),
            pltpu.VMEM((B + 48,), jnp.int32),
            pltpu.VMEM((_WIN,), jnp.int32),
            pltpu.VMEM((_LCAP,), jnp.int32),
            pltpu.VMEM((_LCAP,), jnp.int32),
            pltpu.VMEM((16,), jnp.int32),
        ],
        compiler_params=pltpu.CompilerParams(
            needs_layout_passes=False, use_tc_tiling_on_sc=True),
    )
    def k(idx_hbm, pf_hbm, gf_hbm, cnt_hbm,
          idx_v, comp_v, win_v, pfin_v, gfin1_v, cnt_v):
        iota = lax.iota(jnp.int32, 16)
        wid = lax.axis_index("s") * _NC + lax.axis_index("c")
        lo = wid * SLAB_LO + 8 * jnp.minimum(wid, EXTRA)
        has_extra = wid < EXTRA
        nrows = jnp.where(has_extra, SLAB_HI, SLAB_LO)
        WIN = _WIN
        SENT = _SENT
        LIST = _LCAP - _GRP
        pltpu.sync_copy(idx_hbm, idx_v.at[pl.ds(0, B)])
        neg1 = jnp.full((16,), -1, jnp.int32)

        def init_body(j, _):
            win_v[pl.ds(j * 16, 16)] = neg1
            return 0
        lax.fori_loop(0, WIN // 16, init_body, 0)

        hi = lo + nrows

        one = jnp.full((16,), 1, jnp.int32)
        zero = jnp.full((16,), 0, jnp.int32)

        def filt_body(i, off):
            v = idx_v[pl.ds(i * 16, 16)]
            inm = (v >= lo) & (v < hi)
            inm_i = jnp.where(inm, one, zero)
            comp = (v - lo) * 16384 + (i * 16 + iota)
            cs = plsc.cumsum(inm_i)
            pos = jnp.where(inm, off + cs - 1, B + 32 + iota)
            plsc.store_scatter(comp_v, [pos], comp)
            return off + jnp.sum(inm_i)
        off = lax.fori_loop(0, NIV, filt_body, jnp.int32(0))
        comp_v[pl.ds(off, 16)] = jnp.full((16,), SENT * 16384, jnp.int32)

        lane_masks = [iota == l for l in range(16)]

        def dedup_body(j, _):
            cvec = comp_v[pl.ds(j * 16, 16)]
            loc = lax.shift_right_logical(cvec, 14)
            p = cvec & 16383
            for lm in lane_masks:
                loc_safe = jnp.where(lm, loc, SENT + 16 + iota)
                plsc.store_scatter(win_v, [loc_safe], p)
            return 0
        lax.fori_loop(0, (off + 15) // 16, dedup_body, 0)

        def scan_body(j, cnt):
            w = win_v[pl.ds(j * 16, 16)]
            mk = w >= 0
            mk_i = jnp.where(mk, one, zero)
            g = (lo + j * 16) + iota
            cs = plsc.cumsum(mk_i)
            pos = jnp.where(mk, cnt + cs - 1, LIST + iota)
            plsc.store_scatter(pfin_v, [pos], w)
            plsc.store_scatter(gfin1_v, [pos], g)
            return cnt + jnp.sum(mk_i)
        cnt = lax.fori_loop(0, (SLAB_HI + 15) // 16, scan_body, jnp.int32(0))

        @pl.when(cnt > 0)
        def _():
            def pad_body(c, _):
                pos = cnt + c * 16 + iota
                src = lax.rem(pos, cnt)
                pfin_v[pl.ds(cnt + c * 16, 16)] = plsc.load_gather(
                    pfin_v, [src])
                gfin1_v[pl.ds(cnt + c * 16, 16)] = plsc.load_gather(
                    gfin1_v, [src])
                return 0
            lax.fori_loop(0, _GRP // 16, pad_body, 0)

        cnt_v[pl.ds(0, 16)] = jnp.where(iota == 0, cnt, 0)
        pltpu.sync_copy(pfin_v, pf_hbm.at[wid])
        pltpu.sync_copy(gfin1_v, gf_hbm.at[wid])
        pltpu.sync_copy(cnt_v, cnt_hbm.at[wid])

    return k(idx)


def _scatter_rows(pf, gf, cnts, upd, out_ref, D):
    mesh = plsc.VectorSubcoreMesh(core_axis_name="c", subcore_axis_name="s")

    @functools.partial(
        pl.kernel,
        mesh=mesh,
        out_type=(),
        scratch_types=[
            pltpu.VMEM((_LCAP,), jnp.int32),
            pltpu.VMEM((_LCAP,), jnp.int32),
            pltpu.VMEM((_GRP,), jnp.int32),
            pltpu.VMEM((_GRP,), jnp.int32),
            pltpu.VMEM((_GRP, D), jnp.float32),
            pltpu.VMEM((16,), jnp.int32),
            pltpu.SemaphoreType.DMA,
            pltpu.SemaphoreType.DMA,
        ],
        compiler_params=pltpu.CompilerParams(
            needs_layout_passes=False, use_tc_tiling_on_sc=True),
    )
    def k(pf_hbm, gf_hbm, cnt_hbm, upd_hbm, out_hbm,
          pfin_v, gfin_v, pidx, gidx, rows, cnt_v, sem_g, sem_s):
        wid = lax.axis_index("s") * _NC + lax.axis_index("c")
        pltpu.sync_copy(pf_hbm.at[wid], pfin_v)
        pltpu.sync_copy(gf_hbm.at[wid], gfin_v)
        pltpu.sync_copy(cnt_hbm.at[wid], cnt_v)
        cnt = jnp.sum(cnt_v[pl.ds(0, 16)])

        @pl.when(cnt > 0)
        def _():
            def grp_body(g, _):
                base = g * _GRP
                for t in range(_GRP // 16):
                    pidx[pl.ds(t * 16, 16)] = pfin_v[pl.ds(base + t * 16, 16)]
                    gidx[pl.ds(t * 16, 16)] = gfin_v[pl.ds(base + t * 16, 16)]
                pltpu.async_copy(upd_hbm.at[pidx], rows, sem_g).wait()
                pltpu.async_copy(rows, out_hbm.at[gidx], sem_s).wait()
                return 0
            lax.fori_loop(0, (cnt + _GRP - 1) // _GRP, grp_body, 0)

    k(pf, gf, cnts, upd, out_ref)


def kernel(node_idxs, node_features, edge_features, timestamps, memory, messages, last_update, W1, b1, W2, b2, W_ih, W_hh, b_ih, b_hh):
    M, H = memory.shape
    node_memory = _gather_rows(memory, node_idxs)
    pf, gf, cnts = _index_winners(node_idxs, M)
    mem_copy = _copy_table(memory)
    updated = _dense_update(node_features, node_memory, edge_features,
                            W1, b1, W2, b2, W_ih, W_hh, b_ih, b_hh)
    out_ref = jax.new_ref(mem_copy)
    _scatter_rows(pf, gf, cnts, updated, out_ref, H)
    return out_ref[...]


# speedup vs baseline: 1.3028x; 1.2044x over previous
import functools

import jax
import jax.numpy as jnp
from jax import lax
from jax.experimental import pallas as pl
from jax.experimental.pallas import tpu as pltpu
from jax.experimental.pallas import tpu_sc as plsc

_NC = 2
_NS = 16
_NW = _NC * _NS


def _gather_rows(table, idx):
    V, D = table.shape
    B = idx.shape[0]
    b_per_w = B // _NW
    mesh = plsc.VectorSubcoreMesh(core_axis_name="c", subcore_axis_name="s")

    @functools.partial(
        pl.kernel,
        mesh=mesh,
        out_type=jax.ShapeDtypeStruct((B, D), jnp.float32),
        scratch_types=[
            pltpu.VMEM((b_per_w,), jnp.int32),
            pltpu.VMEM((b_per_w, D), jnp.float32),
            pltpu.SemaphoreType.DMA,
        ],
    )
    def k(table_hbm, idx_hbm, out_hbm, idx_v, rows_v, sem):
        wid = lax.axis_index("s") * _NC + lax.axis_index("c")
        base = wid * b_per_w
        pltpu.sync_copy(idx_hbm.at[pl.ds(base, b_per_w)], idx_v)
        pltpu.async_copy(table_hbm.at[idx_v], rows_v, sem).wait()
        pltpu.sync_copy(rows_v, out_hbm.at[pl.ds(base, b_per_w)])

    return k(table, idx)


def _copy_table(memory):
    M, H = memory.shape
    MBLK = 4000

    def body(mem_ref, out_ref):
        out_ref[...] = mem_ref[...]

    return pl.pallas_call(
        body,
        grid=(M // MBLK,),
        in_specs=[pl.BlockSpec((MBLK, H), lambda i: (i, 0))],
        out_specs=pl.BlockSpec((MBLK, H), lambda i: (i, 0)),
        out_shape=jax.ShapeDtypeStruct((M, H), jnp.float32),
    )(memory)


def _dense_update(nf, nm, ef, W1, b1, W2, b2, W_ih, W_hh, b_ih, b_hh):
    B, H = nm.shape
    G = 16
    BLK = B // G

    b1r = b1.reshape(1, -1)
    b2r = b2.reshape(1, -1)
    b_ihr = b_ih.reshape(1, -1)
    b_hhr = b_hh.reshape(1, -1)

    def body(nf_ref, nm_ref, ef_ref, W1_ref, b1_ref, W2_ref, b2_ref,
             W_ih_ref, W_hh_ref, b_ih_ref, b_hh_ref, out_ref):
        nf_b = nf_ref[...]
        nm_b = nm_ref[...]
        ef_b = ef_ref[...]
        x = jnp.concatenate([nf_b, nm_b, ef_b], axis=1)
        h1 = jnp.maximum(
            jnp.dot(x, W1_ref[...], preferred_element_type=jnp.float32)
            + b1_ref[...], 0.0)
        msgs = jnp.dot(h1, W2_ref[...], preferred_element_type=jnp.float32) + b2_ref[...]
        gx = lax.dot_general(msgs, W_ih_ref[...], (((1,), (1,)), ((), ())),
                             preferred_element_type=jnp.float32) + b_ih_ref[...]
        gh = lax.dot_general(nm_b, W_hh_ref[...], (((1,), (1,)), ((), ())),
                             preferred_element_type=jnp.float32) + b_hh_ref[...]
        r = jax.nn.sigmoid(gx[:, :H] + gh[:, :H])
        z = jax.nn.sigmoid(gx[:, H:2 * H] + gh[:, H:2 * H])
        n = jnp.tanh(gx[:, 2 * H:] + r * gh[:, 2 * H:])
        out_ref[...] = (1.0 - z) * n + z * nm_b

    full = lambda shape: pl.BlockSpec(shape, lambda i: (0,) * len(shape))
    return pl.pallas_call(
        body,
        grid=(G,),
        in_specs=[
            pl.BlockSpec((BLK, H), lambda i: (i, 0)),
            pl.BlockSpec((BLK, H), lambda i: (i, 0)),
            pl.BlockSpec((BLK, H), lambda i: (i, 0)),
            full(W1.shape), full(b1r.shape), full(W2.shape), full(b2r.shape),
            full(W_ih.shape), full(W_hh.shape), full(b_ihr.shape), full(b_hhr.shape),
        ],
        out_specs=pl.BlockSpec((BLK, H), lambda i: (i, 0)),
        out_shape=jax.ShapeDtypeStruct((B, H), jnp.float32),
    )(nf, nm, ef, W1, b1r, W2, b2r, W_ih, W_hh, b_ihr, b_hhr)


_WIN = 3328
_SENT = 3200
_GRP = 768
_LCAP = 4096


def _index_winners(idx, M):
    B = idx.shape[0]
    SLAB_LO = (M // _NW) & ~7
    EXTRA = (M - SLAB_LO * _NW) // 8
    SLAB_HI = SLAB_LO + 8
    NIV = B // 16
    mesh = plsc.VectorSubcoreMesh(core_axis_name="c", subcore_axis_name="s")

    @functools.partial(
        pl.kernel,
        mesh=mesh,
        out_type=(
            jax.ShapeDtypeStruct((_NW, _LCAP), jnp.int32),
            jax.ShapeDtypeStruct((_NW, _LCAP), jnp.int32),
            jax.ShapeDtypeStruct((_NW, 16), jnp.int32),
        ),
        scratch_types=[
            pltpu.VMEM((B + 16,), jnp.int32),
            pltpu.VMEM((B + 48,), jnp.int32),
            pltpu.VMEM((_WIN,), jnp.int32),
            pltpu.VMEM((_LCAP,), jnp.int32),
            pltpu.VMEM((_LCAP,), jnp.int32),
            pltpu.VMEM((16,), jnp.int32),
        ],
        compiler_params=pltpu.CompilerParams(
            needs_layout_passes=False, use_tc_tiling_on_sc=True),
    )
    def k(idx_hbm, pf_hbm, gf_hbm, cnt_hbm,
          idx_v, comp_v, win_v, pfin_v, gfin1_v, cnt_v):
        iota = lax.iota(jnp.int32, 16)
        wid = lax.axis_index("s") * _NC + lax.axis_index("c")
        lo = wid * SLAB_LO + 8 * jnp.minimum(wid, EXTRA)
        has_extra = wid < EXTRA
        nrows = jnp.where(has_extra, SLAB_HI, SLAB_LO)
        WIN = _WIN
        SENT = _SENT
        LIST = _LCAP - _GRP
        pltpu.sync_copy(idx_hbm, idx_v.at[pl.ds(0, B)])
        neg1 = jnp.full((16,), -1, jnp.int32)

        def init_body(j, _):
            win_v[pl.ds(j * 16, 16)] = neg1
            return 0
        lax.fori_loop(0, WIN // 16, init_body, 0)

        hi = lo + nrows

        one = jnp.full((16,), 1, jnp.int32)
        zero = jnp.full((16,), 0, jnp.int32)

        def filt_body(i, off):
            v = idx_v[pl.ds(i * 16, 16)]
            inm = (v >= lo) & (v < hi)
            inm_i = jnp.where(inm, one, zero)
            comp = (v - lo) * 16384 + (i * 16 + iota)
            cs = plsc.cumsum(inm_i)
            pos = jnp.where(inm, off + cs - 1, B + 32 + iota)
            plsc.store_scatter(comp_v, [pos], comp)
            return off + jnp.sum(inm_i)
        off = lax.fori_loop(0, NIV, filt_body, jnp.int32(0))
        comp_v[pl.ds(off, 16)] = jnp.full((16,), SENT * 16384, jnp.int32)

        lane_masks = [iota == l for l in range(16)]

        def dedup_body(j, _):
            cvec = comp_v[pl.ds(j * 16, 16)]
            loc = lax.shift_right_logical(cvec, 14)
            p = cvec & 16383
            for lm in lane_masks:
                loc_safe = jnp.where(lm, loc, SENT + 16 + iota)
                plsc.store_scatter(win_v, [loc_safe], p)
            return 0
        lax.fori_loop(0, (off + 15) // 16, dedup_body, 0)

        def scan_body(j, cnt):
            w = win_v[pl.ds(j * 16, 16)]
            mk = w >= 0
            mk_i = jnp.where(mk, one, zero)
            g = (lo + j * 16) + iota
            cs = plsc.cumsum(mk_i)
            pos = jnp.where(mk, cnt + cs - 1, LIST + iota)
            plsc.store_scatter(pfin_v, [pos], w)
            plsc.store_scatter(gfin1_v, [pos], g)
            return cnt + jnp.sum(mk_i)
        cnt = lax.fori_loop(0, (SLAB_HI + 15) // 16, scan_body, jnp.int32(0))

        @pl.when(cnt > 0)
        def _():
            def pad_body(c, _):
                pos = cnt + c * 16 + iota
                src = lax.rem(pos, cnt)
                pfin_v[pl.ds(cnt + c * 16, 16)] = plsc.load_gather(
                    pfin_v, [src])
                gfin1_v[pl.ds(cnt + c * 16, 16)] = plsc.load_gather(
                    gfin1_v, [src])
                return 0
            lax.fori_loop(0, _GRP // 16, pad_body, 0)

        cnt_v[pl.ds(0, 16)] = jnp.where(iota == 0, cnt, 0)
        pltpu.sync_copy(pfin_v, pf_hbm.at[wid])
        pltpu.sync_copy(gfin1_v, gf_hbm.at[wid])
        pltpu.sync_copy(cnt_v, cnt_hbm.at[wid])

    return k(idx)


def _scatter_rows(pf, gf, cnts, upd, out_ref, D):
    mesh = plsc.VectorSubcoreMesh(core_axis_name="c", subcore_axis_name="s")

    @functools.partial(
        pl.kernel,
        mesh=mesh,
        out_type=(),
        scratch_types=[
            pltpu.VMEM((_LCAP,), jnp.int32),
            pltpu.VMEM((_LCAP,), jnp.int32),
            pltpu.VMEM((_GRP,), jnp.int32),
            pltpu.VMEM((_GRP,), jnp.int32),
            pltpu.VMEM((_GRP, D), jnp.float32),
            pltpu.VMEM((16,), jnp.int32),
            pltpu.SemaphoreType.DMA,
            pltpu.SemaphoreType.DMA,
        ],
        compiler_params=pltpu.CompilerParams(
            needs_layout_passes=False, use_tc_tiling_on_sc=True),
    )
    def k(pf_hbm, gf_hbm, cnt_hbm, upd_hbm, out_hbm,
          pfin_v, gfin_v, pidx, gidx, rows, cnt_v, sem_g, sem_s):
        wid = lax.axis_index("s") * _NC + lax.axis_index("c")
        pltpu.sync_copy(pf_hbm.at[wid], pfin_v)
        pltpu.sync_copy(gf_hbm.at[wid], gfin_v)
        pltpu.sync_copy(cnt_hbm.at[wid], cnt_v)
        cnt = jnp.sum(cnt_v[pl.ds(0, 16)])

        @pl.when(cnt > 0)
        def _():
            def grp_body(g, _):
                base = g * _GRP
                for t in range(_GRP // 16):
                    pidx[pl.ds(t * 16, 16)] = pfin_v[pl.ds(base + t * 16, 16)]
                    gidx[pl.ds(t * 16, 16)] = gfin_v[pl.ds(base + t * 16, 16)]
                pltpu.async_copy(upd_hbm.at[pidx], rows, sem_g).wait()
                pltpu.async_copy(rows, out_hbm.at[gidx], sem_s).wait()
                return 0
            lax.fori_loop(0, (cnt + _GRP - 1) // _GRP, grp_body, 0)

    k(pf, gf, cnts, upd, out_ref)


def kernel(node_idxs, node_features, edge_features, timestamps, memory, messages, last_update, W1, b1, W2, b2, W_ih, W_hh, b_ih, b_hh):
    M, H = memory.shape
    node_memory = _gather_rows(memory, node_idxs)
    pf, gf, cnts = _index_winners(node_idxs, M)
    mem_copy = _copy_table(memory)
    updated = _dense_update(node_features, node_memory, edge_features,
                            W1, b1, W2, b2, W_ih, W_hh, b_ih, b_hh)
    out_ref = jax.new_ref(mem_copy)
    _scatter_rows(pf, gf, cnts, updated, out_ref, H)
    return out_ref[...]


# speedup vs baseline: 1.4133x; 1.0848x over previous
import functools

import jax
import jax.numpy as jnp
from jax import lax
from jax.experimental import pallas as pl
from jax.experimental.pallas import tpu as pltpu
from jax.experimental.pallas import tpu_sc as plsc

_NC = 2
_NS = 16
_NW = _NC * _NS


def _gather_rows(table, idx):
    V, D = table.shape
    B = idx.shape[0]
    b_per_w = B // _NW
    mesh = plsc.VectorSubcoreMesh(core_axis_name="c", subcore_axis_name="s")

    @functools.partial(
        pl.kernel,
        mesh=mesh,
        out_type=jax.ShapeDtypeStruct((B, D), jnp.float32),
        scratch_types=[
            pltpu.VMEM((b_per_w,), jnp.int32),
            pltpu.VMEM((b_per_w, D), jnp.float32),
            pltpu.SemaphoreType.DMA,
        ],
    )
    def k(table_hbm, idx_hbm, out_hbm, idx_v, rows_v, sem):
        wid = lax.axis_index("s") * _NC + lax.axis_index("c")
        base = wid * b_per_w
        pltpu.sync_copy(idx_hbm.at[pl.ds(base, b_per_w)], idx_v)
        pltpu.async_copy(table_hbm.at[idx_v], rows_v, sem).wait()
        pltpu.sync_copy(rows_v, out_hbm.at[pl.ds(base, b_per_w)])

    return k(table, idx)


def _copy_table(memory):
    M, H = memory.shape
    MBLK = 10000

    def body(mem_ref, out_ref):
        out_ref[...] = mem_ref[...]

    return pl.pallas_call(
        body,
        grid=(M // MBLK,),
        in_specs=[pl.BlockSpec((MBLK, H), lambda i: (i, 0))],
        out_specs=pl.BlockSpec((MBLK, H), lambda i: (i, 0)),
        out_shape=jax.ShapeDtypeStruct((M, H), jnp.float32),
    )(memory)


def _dense_update(nf, nm, ef, W1, b1, W2, b2, W_ih, W_hh, b_ih, b_hh):
    B, H = nm.shape
    G = 8
    BLK = B // G

    b1r = b1.reshape(1, -1)
    b2r = b2.reshape(1, -1)
    b_ihr = b_ih.reshape(1, -1)
    b_hhr = b_hh.reshape(1, -1)

    def body(nf_ref, nm_ref, ef_ref, W1_ref, b1_ref, W2_ref, b2_ref,
             W_ih_ref, W_hh_ref, b_ih_ref, b_hh_ref, out_ref):
        nf_b = nf_ref[...]
        nm_b = nm_ref[...]
        ef_b = ef_ref[...]
        x = jnp.concatenate([nf_b, nm_b, ef_b], axis=1)
        h1 = jnp.maximum(
            jnp.dot(x, W1_ref[...], preferred_element_type=jnp.float32)
            + b1_ref[...], 0.0)
        msgs = jnp.dot(h1, W2_ref[...], preferred_element_type=jnp.float32) + b2_ref[...]
        gx = lax.dot_general(msgs, W_ih_ref[...], (((1,), (1,)), ((), ())),
                             preferred_element_type=jnp.float32) + b_ih_ref[...]
        gh = lax.dot_general(nm_b, W_hh_ref[...], (((1,), (1,)), ((), ())),
                             preferred_element_type=jnp.float32) + b_hh_ref[...]
        r = jax.nn.sigmoid(gx[:, :H] + gh[:, :H])
        z = jax.nn.sigmoid(gx[:, H:2 * H] + gh[:, H:2 * H])
        n = jnp.tanh(gx[:, 2 * H:] + r * gh[:, 2 * H:])
        out_ref[...] = (1.0 - z) * n + z * nm_b

    full = lambda shape: pl.BlockSpec(shape, lambda i: (0,) * len(shape))
    return pl.pallas_call(
        body,
        grid=(G,),
        in_specs=[
            pl.BlockSpec((BLK, H), lambda i: (i, 0)),
            pl.BlockSpec((BLK, H), lambda i: (i, 0)),
            pl.BlockSpec((BLK, H), lambda i: (i, 0)),
            full(W1.shape), full(b1r.shape), full(W2.shape), full(b2r.shape),
            full(W_ih.shape), full(W_hh.shape), full(b_ihr.shape), full(b_hhr.shape),
        ],
        out_specs=pl.BlockSpec((BLK, H), lambda i: (i, 0)),
        out_shape=jax.ShapeDtypeStruct((B, H), jnp.float32),
    )(nf, nm, ef, W1, b1r, W2, b2r, W_ih, W_hh, b_ihr, b_hhr)


_WIN = 3328
_SENT = 3200
_GRP = 768
_LCAP = 4096


def _index_winners(idx, M):
    B = idx.shape[0]
    SLAB_LO = (M // _NW) & ~7
    EXTRA = (M - SLAB_LO * _NW) // 8
    SLAB_HI = SLAB_LO + 8
    NIV = B // 16
    mesh = plsc.VectorSubcoreMesh(core_axis_name="c", subcore_axis_name="s")

    @functools.partial(
        pl.kernel,
        mesh=mesh,
        out_type=(
            jax.ShapeDtypeStruct((_NW, _LCAP), jnp.int32),
            jax.ShapeDtypeStruct((_NW, _LCAP), jnp.int32),
            jax.ShapeDtypeStruct((_NW, 16), jnp.int32),
        ),
        scratch_types=[
            pltpu.VMEM((B + 16,), jnp.int32),
            pltpu.VMEM((B + 48,), jnp.int32),
            pltpu.VMEM((_WIN,), jnp.int32),
            pltpu.VMEM((_LCAP,), jnp.int32),
            pltpu.VMEM((_LCAP,), jnp.int32),
            pltpu.VMEM((16,), jnp.int32),
        ],
        compiler_params=pltpu.CompilerParams(
            needs_layout_passes=False, use_tc_tiling_on_sc=True),
    )
    def k(idx_hbm, pf_hbm, gf_hbm, cnt_hbm,
          idx_v, comp_v, win_v, pfin_v, gfin1_v, cnt_v):
        iota = lax.iota(jnp.int32, 16)
        wid = lax.axis_index("s") * _NC + lax.axis_index("c")
        lo = wid * SLAB_LO + 8 * jnp.minimum(wid, EXTRA)
        has_extra = wid < EXTRA
        nrows = jnp.where(has_extra, SLAB_HI, SLAB_LO)
        WIN = _WIN
        SENT = _SENT
        LIST = _LCAP - _GRP
        pltpu.sync_copy(idx_hbm, idx_v.at[pl.ds(0, B)])
        neg1 = jnp.full((16,), -1, jnp.int32)

        def init_body(j, _):
            win_v[pl.ds(j * 16, 16)] = neg1
            return 0
        lax.fori_loop(0, WIN // 16, init_body, 0)

        hi = lo + nrows

        one = jnp.full((16,), 1, jnp.int32)
        zero = jnp.full((16,), 0, jnp.int32)

        def filt_body(i, off):
            v = idx_v[pl.ds(i * 16, 16)]
            inm = (v >= lo) & (v < hi)
            inm_i = jnp.where(inm, one, zero)
            comp = (v - lo) * 16384 + (i * 16 + iota)
            cs = plsc.cumsum(inm_i)
            pos = jnp.where(inm, off + cs - 1, B + 32 + iota)
            plsc.store_scatter(comp_v, [pos], comp)
            return off + jnp.sum(inm_i)
        off = lax.fori_loop(0, NIV, filt_body, jnp.int32(0))
        comp_v[pl.ds(off, 16)] = jnp.full((16,), SENT * 16384, jnp.int32)

        lane_masks = [iota == l for l in range(16)]

        def dedup_body(j, _):
            cvec = comp_v[pl.ds(j * 16, 16)]
            loc = lax.shift_right_logical(cvec, 14)
            p = cvec & 16383
            for lm in lane_masks:
                loc_safe = jnp.where(lm, loc, SENT + 16 + iota)
                plsc.store_scatter(win_v, [loc_safe], p)
            return 0
        lax.fori_loop(0, (off + 15) // 16, dedup_body, 0)

        def scan_body(j, cnt):
            w = win_v[pl.ds(j * 16, 16)]
            mk = w >= 0
            mk_i = jnp.where(mk, one, zero)
            g = (lo + j * 16) + iota
            cs = plsc.cumsum(mk_i)
            pos = jnp.where(mk, cnt + cs - 1, LIST + iota)
            plsc.store_scatter(pfin_v, [pos], w)
            plsc.store_scatter(gfin1_v, [pos], g)
            return cnt + jnp.sum(mk_i)
        cnt = lax.fori_loop(0, (SLAB_HI + 15) // 16, scan_body, jnp.int32(0))

        @pl.when(cnt > 0)
        def _():
            def pad_body(c, _):
                pos = cnt + c * 16 + iota
                src = lax.rem(pos, cnt)
                pfin_v[pl.ds(cnt + c * 16, 16)] = plsc.load_gather(
                    pfin_v, [src])
                gfin1_v[pl.ds(cnt + c * 16, 16)] = plsc.load_gather(
                    gfin1_v, [src])
                return 0
            lax.fori_loop(0, _GRP // 16, pad_body, 0)

        cnt_v[pl.ds(0, 16)] = jnp.where(iota == 0, cnt, 0)
        pltpu.sync_copy(pfin_v, pf_hbm.at[wid])
        pltpu.sync_copy(gfin1_v, gf_hbm.at[wid])
        pltpu.sync_copy(cnt_v, cnt_hbm.at[wid])

    return k(idx)


def _scatter_rows(pf, gf, cnts, upd, out_ref, D):
    mesh = plsc.VectorSubcoreMesh(core_axis_name="c", subcore_axis_name="s")

    @functools.partial(
        pl.kernel,
        mesh=mesh,
        out_type=(),
        scratch_types=[
            pltpu.VMEM((_LCAP,), jnp.int32),
            pltpu.VMEM((_LCAP,), jnp.int32),
            pltpu.VMEM((_GRP,), jnp.int32),
            pltpu.VMEM((_GRP,), jnp.int32),
            pltpu.VMEM((_GRP, D), jnp.float32),
            pltpu.VMEM((16,), jnp.int32),
            pltpu.SemaphoreType.DMA,
            pltpu.SemaphoreType.DMA,
        ],
        compiler_params=pltpu.CompilerParams(
            needs_layout_passes=False, use_tc_tiling_on_sc=True),
    )
    def k(pf_hbm, gf_hbm, cnt_hbm, upd_hbm, out_hbm,
          pfin_v, gfin_v, pidx, gidx, rows, cnt_v, sem_g, sem_s):
        wid = lax.axis_index("s") * _NC + lax.axis_index("c")
        pltpu.sync_copy(pf_hbm.at[wid], pfin_v)
        pltpu.sync_copy(gf_hbm.at[wid], gfin_v)
        pltpu.sync_copy(cnt_hbm.at[wid], cnt_v)
        cnt = jnp.sum(cnt_v[pl.ds(0, 16)])

        @pl.when(cnt > 0)
        def _():
            def grp_body(g, _):
                base = g * _GRP
                for t in range(_GRP // 16):
                    pidx[pl.ds(t * 16, 16)] = pfin_v[pl.ds(base + t * 16, 16)]
                    gidx[pl.ds(t * 16, 16)] = gfin_v[pl.ds(base + t * 16, 16)]
                pltpu.async_copy(upd_hbm.at[pidx], rows, sem_g).wait()
                pltpu.async_copy(rows, out_hbm.at[gidx], sem_s).wait()
                return 0
            lax.fori_loop(0, (cnt + _GRP - 1) // _GRP, grp_body, 0)

    k(pf, gf, cnts, upd, out_ref)


def kernel(node_idxs, node_features, edge_features, timestamps, memory, messages, last_update, W1, b1, W2, b2, W_ih, W_hh, b_ih, b_hh):
    M, H = memory.shape
    node_memory = _gather_rows(memory, node_idxs)
    pf, gf, cnts = _index_winners(node_idxs, M)
    mem_copy = _copy_table(memory)
    updated = _dense_update(node_features, node_memory, edge_features,
                            W1, b1, W2, b2, W_ih, W_hh, b_ih, b_hh)
    out_ref = jax.new_ref(mem_copy)
    _scatter_rows(pf, gf, cnts, updated, out_ref, H)
    return out_ref[...]


# speedup vs baseline: 1.4326x; 1.0137x over previous
import functools

import jax
import jax.numpy as jnp
from jax import lax
from jax.experimental import pallas as pl
from jax.experimental.pallas import tpu as pltpu
from jax.experimental.pallas import tpu_sc as plsc

_NC = 2
_NS = 16
_NW = _NC * _NS


def _gather_rows(table, idx):
    V, D = table.shape
    B = idx.shape[0]
    b_per_w = B // _NW
    mesh = plsc.VectorSubcoreMesh(core_axis_name="c", subcore_axis_name="s")

    @functools.partial(
        pl.kernel,
        mesh=mesh,
        out_type=jax.ShapeDtypeStruct((B, D), jnp.float32),
        scratch_types=[
            pltpu.VMEM((b_per_w,), jnp.int32),
            pltpu.VMEM((b_per_w, D), jnp.float32),
            pltpu.SemaphoreType.DMA,
        ],
    )
    def k(table_hbm, idx_hbm, out_hbm, idx_v, rows_v, sem):
        wid = lax.axis_index("s") * _NC + lax.axis_index("c")
        base = wid * b_per_w
        pltpu.sync_copy(idx_hbm.at[pl.ds(base, b_per_w)], idx_v)
        pltpu.async_copy(table_hbm.at[idx_v], rows_v, sem).wait()
        pltpu.sync_copy(rows_v, out_hbm.at[pl.ds(base, b_per_w)])

    return k(table, idx)


def _copy_table(memory):
    M, H = memory.shape
    MBLK = 20000

    def body(mem_ref, out_ref):
        out_ref[...] = mem_ref[...]

    return pl.pallas_call(
        body,
        grid=(M // MBLK,),
        in_specs=[pl.BlockSpec((MBLK, H), lambda i: (i, 0))],
        out_specs=pl.BlockSpec((MBLK, H), lambda i: (i, 0)),
        out_shape=jax.ShapeDtypeStruct((M, H), jnp.float32),
    )(memory)


def _dense_update(nf, nm, ef, W1, b1, W2, b2, W_ih, W_hh, b_ih, b_hh):
    B, H = nm.shape
    G = 4
    BLK = B // G

    b1r = b1.reshape(1, -1)
    b2r = b2.reshape(1, -1)
    b_ihr = b_ih.reshape(1, -1)
    b_hhr = b_hh.reshape(1, -1)

    def body(nf_ref, nm_ref, ef_ref, W1_ref, b1_ref, W2_ref, b2_ref,
             W_ih_ref, W_hh_ref, b_ih_ref, b_hh_ref, out_ref):
        nf_b = nf_ref[...]
        nm_b = nm_ref[...]
        ef_b = ef_ref[...]
        x = jnp.concatenate([nf_b, nm_b, ef_b], axis=1)
        h1 = jnp.maximum(
            jnp.dot(x, W1_ref[...], preferred_element_type=jnp.float32)
            + b1_ref[...], 0.0)
        msgs = jnp.dot(h1, W2_ref[...], preferred_element_type=jnp.float32) + b2_ref[...]
        gx = lax.dot_general(msgs, W_ih_ref[...], (((1,), (1,)), ((), ())),
                             preferred_element_type=jnp.float32) + b_ih_ref[...]
        gh = lax.dot_general(nm_b, W_hh_ref[...], (((1,), (1,)), ((), ())),
                             preferred_element_type=jnp.float32) + b_hh_ref[...]
        r = jax.nn.sigmoid(gx[:, :H] + gh[:, :H])
        z = jax.nn.sigmoid(gx[:, H:2 * H] + gh[:, H:2 * H])
        n = jnp.tanh(gx[:, 2 * H:] + r * gh[:, 2 * H:])
        out_ref[...] = (1.0 - z) * n + z * nm_b

    full = lambda shape: pl.BlockSpec(shape, lambda i: (0,) * len(shape))
    return pl.pallas_call(
        body,
        grid=(G,),
        in_specs=[
            pl.BlockSpec((BLK, H), lambda i: (i, 0)),
            pl.BlockSpec((BLK, H), lambda i: (i, 0)),
            pl.BlockSpec((BLK, H), lambda i: (i, 0)),
            full(W1.shape), full(b1r.shape), full(W2.shape), full(b2r.shape),
            full(W_ih.shape), full(W_hh.shape), full(b_ihr.shape), full(b_hhr.shape),
        ],
        out_specs=pl.BlockSpec((BLK, H), lambda i: (i, 0)),
        out_shape=jax.ShapeDtypeStruct((B, H), jnp.float32),
    )(nf, nm, ef, W1, b1r, W2, b2r, W_ih, W_hh, b_ihr, b_hhr)


_WIN = 3328
_SENT = 3200
_GRP = 768
_LCAP = 4096


def _index_winners(idx, M):
    B = idx.shape[0]
    SLAB_LO = (M // _NW) & ~7
    EXTRA = (M - SLAB_LO * _NW) // 8
    SLAB_HI = SLAB_LO + 8
    NIV = B // 16
    mesh = plsc.VectorSubcoreMesh(core_axis_name="c", subcore_axis_name="s")

    @functools.partial(
        pl.kernel,
        mesh=mesh,
        out_type=(
            jax.ShapeDtypeStruct((_NW, _LCAP), jnp.int32),
            jax.ShapeDtypeStruct((_NW, _LCAP), jnp.int32),
            jax.ShapeDtypeStruct((_NW, 16), jnp.int32),
        ),
        scratch_types=[
            pltpu.VMEM((B + 16,), jnp.int32),
            pltpu.VMEM((B + 48,), jnp.int32),
            pltpu.VMEM((_WIN,), jnp.int32),
            pltpu.VMEM((_LCAP,), jnp.int32),
            pltpu.VMEM((_LCAP,), jnp.int32),
            pltpu.VMEM((16,), jnp.int32),
        ],
        compiler_params=pltpu.CompilerParams(
            needs_layout_passes=False, use_tc_tiling_on_sc=True),
    )
    def k(idx_hbm, pf_hbm, gf_hbm, cnt_hbm,
          idx_v, comp_v, win_v, pfin_v, gfin1_v, cnt_v):
        iota = lax.iota(jnp.int32, 16)
        wid = lax.axis_index("s") * _NC + lax.axis_index("c")
        lo = wid * SLAB_LO + 8 * jnp.minimum(wid, EXTRA)
        has_extra = wid < EXTRA
        nrows = jnp.where(has_extra, SLAB_HI, SLAB_LO)
        WIN = _WIN
        SENT = _SENT
        LIST = _LCAP - _GRP
        pltpu.sync_copy(idx_hbm, idx_v.at[pl.ds(0, B)])
        neg1 = jnp.full((16,), -1, jnp.int32)

        def init_body(j, _):
            win_v[pl.ds(j * 16, 16)] = neg1
            return 0
        lax.fori_loop(0, WIN // 16, init_body, 0)

        hi = lo + nrows

        one = jnp.full((16,), 1, jnp.int32)
        zero = jnp.full((16,), 0, jnp.int32)

        def filt_body(i, off):
            v = idx_v[pl.ds(i * 16, 16)]
            inm = (v >= lo) & (v < hi)
            inm_i = jnp.where(inm, one, zero)
            comp = (v - lo) * 16384 + (i * 16 + iota)
            cs = plsc.cumsum(inm_i)
            pos = jnp.where(inm, off + cs - 1, B + 32 + iota)
            plsc.store_scatter(comp_v, [pos], comp)
            return off + jnp.sum(inm_i)
        off = lax.fori_loop(0, NIV, filt_body, jnp.int32(0))
        comp_v[pl.ds(off, 16)] = jnp.full((16,), SENT * 16384, jnp.int32)

        lane_masks = [iota == l for l in range(16)]

        def dedup_body(j, _):
            cvec = comp_v[pl.ds(j * 16, 16)]
            loc = lax.shift_right_logical(cvec, 14)
            p = cvec & 16383
            for lm in lane_masks:
                loc_safe = jnp.where(lm, loc, SENT + 16 + iota)
                plsc.store_scatter(win_v, [loc_safe], p)
            return 0
        lax.fori_loop(0, (off + 15) // 16, dedup_body, 0)

        def scan_body(j, cnt):
            w = win_v[pl.ds(j * 16, 16)]
            mk = w >= 0
            mk_i = jnp.where(mk, one, zero)
            g = (lo + j * 16) + iota
            cs = plsc.cumsum(mk_i)
            pos = jnp.where(mk, cnt + cs - 1, LIST + iota)
            plsc.store_scatter(pfin_v, [pos], w)
            plsc.store_scatter(gfin1_v, [pos], g)
            return cnt + jnp.sum(mk_i)
        cnt = lax.fori_loop(0, (SLAB_HI + 15) // 16, scan_body, jnp.int32(0))

        @pl.when(cnt > 0)
        def _():
            def pad_body(c, _):
                pos = cnt + c * 16 + iota
                src = lax.rem(pos, cnt)
                pfin_v[pl.ds(cnt + c * 16, 16)] = plsc.load_gather(
                    pfin_v, [src])
                gfin1_v[pl.ds(cnt + c * 16, 16)] = plsc.load_gather(
                    gfin1_v, [src])
                return 0
            lax.fori_loop(0, _GRP // 16, pad_body, 0)

        cnt_v[pl.ds(0, 16)] = jnp.where(iota == 0, cnt, 0)
        pltpu.sync_copy(pfin_v, pf_hbm.at[wid])
        pltpu.sync_copy(gfin1_v, gf_hbm.at[wid])
        pltpu.sync_copy(cnt_v, cnt_hbm.at[wid])

    return k(idx)


def _scatter_rows(pf, gf, cnts, upd, out_ref, D):
    mesh = plsc.VectorSubcoreMesh(core_axis_name="c", subcore_axis_name="s")

    @functools.partial(
        pl.kernel,
        mesh=mesh,
        out_type=(),
        scratch_types=[
            pltpu.VMEM((_LCAP,), jnp.int32),
            pltpu.VMEM((_LCAP,), jnp.int32),
            pltpu.VMEM((_GRP,), jnp.int32),
            pltpu.VMEM((_GRP,), jnp.int32),
            pltpu.VMEM((_GRP, D), jnp.float32),
            pltpu.VMEM((16,), jnp.int32),
            pltpu.SemaphoreType.DMA,
            pltpu.SemaphoreType.DMA,
        ],
        compiler_params=pltpu.CompilerParams(
            needs_layout_passes=False, use_tc_tiling_on_sc=True),
    )
    def k(pf_hbm, gf_hbm, cnt_hbm, upd_hbm, out_hbm,
          pfin_v, gfin_v, pidx, gidx, rows, cnt_v, sem_g, sem_s):
        wid = lax.axis_index("s") * _NC + lax.axis_index("c")
        pltpu.sync_copy(pf_hbm.at[wid], pfin_v)
        pltpu.sync_copy(gf_hbm.at[wid], gfin_v)
        pltpu.sync_copy(cnt_hbm.at[wid], cnt_v)
        cnt = jnp.sum(cnt_v[pl.ds(0, 16)])

        @pl.when(cnt > 0)
        def _():
            def grp_body(g, _):
                base = g * _GRP
                for t in range(_GRP // 16):
                    pidx[pl.ds(t * 16, 16)] = pfin_v[pl.ds(base + t * 16, 16)]
                    gidx[pl.ds(t * 16, 16)] = gfin_v[pl.ds(base + t * 16, 16)]
                pltpu.async_copy(upd_hbm.at[pidx], rows, sem_g).wait()
                pltpu.async_copy(rows, out_hbm.at[gidx], sem_s).wait()
                return 0
            lax.fori_loop(0, (cnt + _GRP - 1) // _GRP, grp_body, 0)

    k(pf, gf, cnts, upd, out_ref)


def kernel(node_idxs, node_features, edge_features, timestamps, memory, messages, last_update, W1, b1, W2, b2, W_ih, W_hh, b_ih, b_hh):
    M, H = memory.shape
    node_memory = _gather_rows(memory, node_idxs)
    pf, gf, cnts = _index_winners(node_idxs, M)
    mem_copy = _copy_table(memory)
    updated = _dense_update(node_features, node_memory, edge_features,
                            W1, b1, W2, b2, W_ih, W_hh, b_ih, b_hh)
    out_ref = jax.new_ref(mem_copy)
    _scatter_rows(pf, gf, cnts, updated, out_ref, H)
    return out_ref[...]
